# baseline (speedup 1.0000x reference)
import functools

import jax
import jax.numpy as jnp
from jax import lax
from jax.experimental import pallas as pl
from jax.experimental.pallas import tpu as pltpu
from jax.experimental.pallas import tpu_sc as plsc

_N = 10000
_E = 160000
_D = 256
_MSG = 128
_OUT = 128
_ACT = 8
_NA = 1000

_NC = 2
_NS = 16
_NW = _NC * _NS


_GSUB = 5
_GCH = _GSUB * 128
_PKW = 128


def _sc_gather(p, q, ei3, base, gnch):
    ec = gnch * _GCH
    gmaxit = (gnch + _NS - 1) // _NS
    mesh = plsc.VectorSubcoreMesh(core_axis_name="c", subcore_axis_name="s")

    @functools.partial(
        pl.kernel,
        mesh=mesh,
        out_type=[
            jax.ShapeDtypeStruct((ec, _PKW), jnp.float32),
            jax.ShapeDtypeStruct((ec, _PKW), jnp.float32),
        ],
        scratch_types=[
            pltpu.VMEM_SHARED((_N, _PKW), jnp.float32),
            pltpu.VMEM((_GSUB, 128), jnp.int32),
            pltpu.VMEM((2, 128, _PKW), jnp.float32),
            pltpu.SemaphoreType.DMA,
            pltpu.SemaphoreType.DMA,
        ],
    )
    def gather_kernel(p_hbm, q_hbm, ei_hbm, ps_hbm, qr_hbm,
                      table, idx_v, buf, sem_g, sem_w):
        c = lax.axis_index("c")
        s = lax.axis_index("s")

        @pl.loop(0, _ZMAXIT)
        def _(jj):
            ch = s + _NS * jj

            @pl.when(ch < _ZNCH)
            def _():
                sl = pl.ds(ch * _ZROWS, _ZROWS)

                @pl.when(c == 0)
                def _():
                    pltpu.sync_copy(p_hbm.at[sl], table.at[sl])

                @pl.when(c == 1)
                def _():
                    pltpu.sync_copy(q_hbm.at[sl], table.at[sl])

        plsc.subcore_barrier()

        def chunk(k, which, out_hbm):
            pltpu.sync_copy(ei_hbm.at[which, base + k], idx_v)
            writes = [None, None]
            for j in range(_GSUB):
                b = j % 2
                if writes[b] is not None:
                    writes[b].wait()
                pltpu.async_copy(table.at[idx_v.at[j]], buf.at[b],
                                 sem_g).wait()
                writes[b] = pltpu.async_copy(
                    buf.at[b], out_hbm.at[pl.ds(k * _GCH + j * 128, 128)],
                    sem_w)
            writes[0].wait()
            writes[1].wait()

        @pl.loop(0, gmaxit)
        def _(jj):
            k = s + _NS * jj

            @pl.when(k < gnch)
            def _():
                @pl.when(c == 0)
                def _():
                    chunk(k, 0, ps_hbm)

                @pl.when(c == 1)
                def _():
                    chunk(k, 1, qr_hbm)

    return gather_kernel(p, q, ei3)


_SSUB = 5
_SCH = _SSUB * 128
_ZROWS = 80
_ZNCH = _N // _ZROWS
_ZMAXIT = (_ZNCH + _NS - 1) // _NS


def _sc_scatter_add(parts, ei3, zrows):
    nparts = len(parts)
    bases = [b for (_, b, _) in parts]
    snchs = [n for (_, _, n) in parts]
    mesh = plsc.VectorSubcoreMesh(core_axis_name="c", subcore_axis_name="s")

    @functools.partial(
        pl.kernel,
        mesh=mesh,
        out_type=jax.ShapeDtypeStruct((_NC, _N, _MSG), jnp.float32),
        scratch_types=[
            pltpu.VMEM((_SSUB, 128), jnp.int32),
            pltpu.VMEM((2, 128, _MSG), jnp.float32),
            pltpu.VMEM((_ZROWS, _MSG), jnp.float32),
            pltpu.VMEM_SHARED((_N, _MSG), jnp.float32),
            pltpu.SemaphoreType.DMA,
        ],
    )
    def scatter_kernel(*args):
        m_hbms = args[:nparts]
        ei_hbm, z_hbm, out_hbm, idx_v, mbuf, zbuf, acc, sem_m = args[nparts:]
        c = lax.axis_index("c")
        s = lax.axis_index("s")

        pltpu.sync_copy(z_hbm, zbuf)

        @pl.loop(0, _ZMAXIT)
        def _(jj):
            ch = s + _NS * jj

            @pl.when(ch < _ZNCH)
            def _():
                pltpu.sync_copy(zbuf, acc.at[pl.ds(ch * _ZROWS, _ZROWS)])

        plsc.subcore_barrier()

        for m_hbm, base, snch in zip(m_hbms, bases, snchs):
            hmaxit = (snch // 2 + _NS - 1) // _NS

            @pl.loop(0, hmaxit)
            def _(jj, m_hbm=m_hbm, base=base, snch=snch):
                t = 2 * (s + _NS * jj) + c

                @pl.when(t < snch)
                def _():
                    pltpu.sync_copy(ei_hbm.at[1, base + t], idx_v)
                    loads = [None, None]
                    loads[0] = pltpu.async_copy(
                        m_hbm.at[pl.ds(t * _SCH, 128)], mbuf.at[0], sem_m)
                    for j in range(_SSUB):
                        if j + 1 < _SSUB:
                            loads[(j + 1) % 2] = pltpu.async_copy(
                                m_hbm.at[pl.ds(t * _SCH + (j + 1) * 128, 128)],
                                mbuf.at[(j + 1) % 2], sem_m)
                        loads[j % 2].wait()
                        pltpu.sync_copy(mbuf.at[j % 2],
                                        acc.at[idx_v.at[j]], add=True)

        plsc.subcore_barrier()

        @pl.loop(0, _ZMAXIT)
        def _(jj):
            ch = s + _NS * jj

            @pl.when(ch < _ZNCH)
            def _():
                pltpu.sync_copy(acc.at[pl.ds(ch * _ZROWS, _ZROWS)],
                                out_hbm.at[c, pl.ds(ch * _ZROWS, _ZROWS)])

    return scatter_kernel(*[m for (m, _, _) in parts], ei3, zrows)


def _pack_pair(v16):
    vb = lax.bitcast_convert_type(v16, jnp.uint16)
    w = (vb[:, :128].astype(jnp.uint32)
         | (vb[:, 128:].astype(jnp.uint32) << 16))
    return lax.bitcast_convert_type(w, jnp.float32)


def _unpack_pair(wf):
    w = lax.bitcast_convert_type(wf, jnp.uint32)
    lo = lax.bitcast_convert_type((w & 0xFFFF).astype(jnp.uint16), jnp.bfloat16)
    hi = lax.bitcast_convert_type((w >> 16).astype(jnp.uint16), jnp.bfloat16)
    return lo, hi


def _pq_body(x_ref, w1a_ref, w1b_ref, b1_ref, p_ref, q_ref):
    xb = x_ref[...].astype(jnp.bfloat16)
    p = (jnp.dot(xb, w1a_ref[...], preferred_element_type=jnp.float32)
         + b1_ref[...])
    q = jnp.dot(xb, w1b_ref[...], preferred_element_type=jnp.float32)
    p_ref[...] = _pack_pair(p.astype(jnp.bfloat16))
    q_ref[...] = _pack_pair(q.astype(jnp.bfloat16))


def _edge_body(ps_ref, qr_ref, w2a_ref, w2b_ref, b2_ref, w3_ref, b3_ref, m_ref):
    ps_a, ps_b = _unpack_pair(ps_ref[...])
    qr_a, qr_b = _unpack_pair(qr_ref[...])
    h1a = jnp.maximum(ps_a + qr_a, 0)
    h1b = jnp.maximum(ps_b + qr_b, 0)
    h2 = (jnp.dot(h1a, w2a_ref[...], preferred_element_type=jnp.float32)
          + jnp.dot(h1b, w2b_ref[...], preferred_element_type=jnp.float32)
          + b2_ref[...])
    h2 = jnp.maximum(h2, 0.0).astype(jnp.bfloat16)
    m_ref[...] = (jnp.dot(h2, w3_ref[...], preferred_element_type=jnp.float32)
                  + b3_ref[...])


def _node_body(ag0_ref, ag1_ref, x_ref, wa1_ref, ba1_ref, wa2_ref, ba2_ref,
               wu1a_ref, wu1b_ref, bu1_ref, wu2_ref, bu2_ref,
               wu3_ref, bu3_ref, wh1_ref, bh1_ref, wh2_ref, bh2_ref,
               wm_ref, bm_ref, wl_ref, bl_ref, out_ref, act_ref):
    aggr = (jnp.sum(ag0_ref[...], axis=0)
            + jnp.sum(ag1_ref[...], axis=0)).astype(jnp.bfloat16)
    a = jnp.maximum(
        jnp.dot(aggr, wa1_ref[...], preferred_element_type=jnp.float32)
        + ba1_ref[...], 0.0).astype(jnp.bfloat16)
    a = jnp.maximum(
        jnp.dot(a, wa2_ref[...], preferred_element_type=jnp.float32)
        + ba2_ref[...], 0.0).astype(jnp.bfloat16)
    h = jnp.maximum(
        jnp.dot(x_ref[...].astype(jnp.bfloat16), wu1a_ref[...],
                preferred_element_type=jnp.float32)
        + jnp.dot(a, wu1b_ref[...], preferred_element_type=jnp.float32)
        + bu1_ref[...], 0.0).astype(jnp.bfloat16)
    h = jnp.maximum(
        jnp.dot(h, wu2_ref[...], preferred_element_type=jnp.float32)
        + bu2_ref[...], 0.0).astype(jnp.bfloat16)
    nodes = (jnp.dot(h, wu3_ref[...], preferred_element_type=jnp.float32)
             + bu3_ref[...])
    out_ref[...] = nodes

    @pl.when(pl.program_id(0) == 0)
    def _():
        z = jnp.maximum(
            jnp.dot(nodes[:_NA], wh1_ref[...],
                    preferred_element_type=jnp.float32) + bh1_ref[...], 0.0)
        z = jnp.maximum(
            jnp.dot(z, wh2_ref[...], preferred_element_type=jnp.float32)
            + bh2_ref[...], 0.0)
        mean = (jnp.dot(z, wm_ref[...], preferred_element_type=jnp.float32)
                + bm_ref[...])
        ls = jnp.clip(
            jnp.dot(z, wl_ref[...], preferred_element_type=jnp.float32)
            + bl_ref[...], -20.0, 2.0)
        act_ref[...] = jnp.concatenate([mean, jnp.exp(ls)], axis=-1)


def _full(shape):
    return pl.BlockSpec(shape, lambda *a: tuple(0 for _ in shape))


_BN = 2000
_BE = 2000


def _tc_pq(x, w1a, w1b, b1):
    grid = (_N // _BN,)
    return pl.pallas_call(
        _pq_body,
        grid=grid,
        in_specs=[
            pl.BlockSpec((_BN, _D), lambda i: (i, 0)),
            _full((_D, 256)),
            _full((_D, 256)),
            _full((1, 256)),
        ],
        out_specs=[
            pl.BlockSpec((_BN, _PKW), lambda i: (i, 0)),
            pl.BlockSpec((_BN, _PKW), lambda i: (i, 0)),
        ],
        out_shape=[
            jax.ShapeDtypeStruct((_N, _PKW), jnp.float32),
            jax.ShapeDtypeStruct((_N, _PKW), jnp.float32),
        ],
    )(x, w1a, w1b, b1)


def _tc_edge_mlp(ps, qr, w2a, w2b, b2, w3, b3):
    grid = (ps.shape[0] // _BE,)
    return pl.pallas_call(
        _edge_body,
        grid=grid,
        in_specs=[
            pl.BlockSpec((_BE, _PKW), lambda i: (i, 0)),
            pl.BlockSpec((_BE, _PKW), lambda i: (i, 0)),
            _full((128, 256)),
            _full((128, 256)),
            _full((1, 256)),
            _full((256, _MSG)),
            _full((1, _MSG)),
        ],
        out_specs=pl.BlockSpec((_BE, _MSG), lambda i: (i, 0)),
        out_shape=jax.ShapeDtypeStruct((ps.shape[0], _MSG), jnp.float32),
    )(ps, qr, w2a, w2b, b2, w3, b3)


def _tc_node_mlp(ag0, ag1, x, wa1, ba1, wa2, ba2, wu1a, wu1b, bu1, wu2, bu2,
                 wu3, bu3, wh1, bh1, wh2, bh2, wm, bm, wl, bl):
    grid = (_N // _BN,)
    return pl.pallas_call(
        _node_body,
        grid=grid,
        in_specs=[
            pl.BlockSpec((_NC, _BN, _MSG), lambda i: (0, i, 0)),
            pl.BlockSpec((_NC, _BN, _MSG), lambda i: (0, i, 0)),
            pl.BlockSpec((_BN, _D), lambda i: (i, 0)),
            _full((_MSG, 128)),
            _full((1, 128)),
            _full((128, 128)),
            _full((1, 128)),
            _full((_D, 256)),
            _full((128, 256)),
            _full((1, 256)),
            _full((256, 256)),
            _full((1, 256)),
            _full((256, _OUT)),
            _full((1, _OUT)),
            _full((_OUT, 256)),
            _full((1, 256)),
            _full((256, 256)),
            _full((1, 256)),
            _full((256, _ACT)),
            _full((1, _ACT)),
            _full((256, _ACT)),
            _full((1, _ACT)),
        ],
        out_specs=[
            pl.BlockSpec((_BN, _OUT), lambda i: (i, 0)),
            _full((_NA, 2 * _ACT)),
        ],
        out_shape=[
            jax.ShapeDtypeStruct((_N, _OUT), jnp.float32),
            jax.ShapeDtypeStruct((_NA, 2 * _ACT), jnp.float32),
        ],
    )(ag0, ag1, x, wa1, ba1, wa2, ba2, wu1a, wu1b, bu1, wu2, bu2, wu3, bu3,
      wh1, bh1, wh2, bh2, wm, bm, wl, bl)


def kernel(x, edge_index, Wm1, bm1, Wm2, bm2, Wm3, bm3, Wa1, ba1, Wa2, ba2,
           Wu1, bu1, Wu2, bu2, Wu3, bu3, Wh1, bh1, Wh2, bh2,
           Wmean, bmean, Wls, bls):
    ei3 = edge_index.reshape(2, _E // _GCH, _GSUB, 128)

    bf = jnp.bfloat16
    w1a, w1b = Wm1[:_D].astype(bf), Wm1[_D:].astype(bf)
    wu1a, wu1b = Wu1[:_D].astype(bf), Wu1[_D:].astype(bf)

    p, q = _tc_pq(x, w1a, w1b, bm1.reshape(1, -1))
    zrows = jnp.zeros((_ZROWS, _MSG), jnp.float32)
    w2a = Wm2[:128].astype(bf)
    w2b = Wm2[128:].astype(bf)
    w3 = Wm3.astype(bf)

    nch = _E // _GCH
    npart = 5
    pch = nch // npart
    ms = []
    for k in range(npart):
        ps, qr = _sc_gather(p, q, ei3, k * pch, pch)
        ms.append((_tc_edge_mlp(ps, qr, w2a, w2b, bm2.reshape(1, -1),
                                w3, bm3.reshape(1, -1)), k * pch, pch))
    ag0 = _sc_scatter_add(ms[:3], ei3, zrows)
    ag1 = _sc_scatter_add(ms[3:], ei3, zrows)
    nodes, act = _tc_node_mlp(
        ag0, ag1, x, Wa1.astype(bf), ba1.reshape(1, -1),
        Wa2.astype(bf), ba2.reshape(1, -1),
        wu1a, wu1b, bu1.reshape(1, -1),
        Wu2.astype(bf), bu2.reshape(1, -1),
        Wu3.astype(bf), bu3.reshape(1, -1),
        Wh1, bh1.reshape(1, -1), Wh2, bh2.reshape(1, -1),
        Wmean, bmean.reshape(1, -1), Wls, bls.reshape(1, -1))
    return act


# speedup vs baseline: 1.1508x; 1.1508x over previous
import functools

import jax
import jax.numpy as jnp
from jax import lax
from jax.experimental import pallas as pl
from jax.experimental.pallas import tpu as pltpu
from jax.experimental.pallas import tpu_sc as plsc

_N = 10000
_E = 160000
_D = 256
_MSG = 128
_OUT = 128
_ACT = 8
_NA = 1000

_NC = 2
_NS = 16
_NW = _NC * _NS


_GSUB = 5
_GCH = _GSUB * 128
_PKW = 128


def _sc_gather(p, q, ei3, base, gnch):
    ec = gnch * _GCH
    gmaxit = (gnch + _NS - 1) // _NS
    mesh = plsc.VectorSubcoreMesh(core_axis_name="c", subcore_axis_name="s")

    @functools.partial(
        pl.kernel,
        mesh=mesh,
        out_type=[
            jax.ShapeDtypeStruct((ec, _PKW), jnp.float32),
            jax.ShapeDtypeStruct((ec, _PKW), jnp.float32),
        ],
        scratch_types=[
            pltpu.VMEM_SHARED((_N, _PKW), jnp.float32),
            pltpu.VMEM((_GSUB, 128), jnp.int32),
            pltpu.VMEM((2, 128, _PKW), jnp.float32),
            pltpu.SemaphoreType.DMA,
            pltpu.SemaphoreType.DMA,
        ],
    )
    def gather_kernel(p_hbm, q_hbm, ei_hbm, ps_hbm, qr_hbm,
                      table, idx_v, buf, sem_g, sem_w):
        c = lax.axis_index("c")
        s = lax.axis_index("s")

        @pl.loop(0, _ZMAXIT)
        def _(jj):
            ch = s + _NS * jj

            @pl.when(ch < _ZNCH)
            def _():
                sl = pl.ds(ch * _ZROWS, _ZROWS)

                @pl.when(c == 0)
                def _():
                    pltpu.sync_copy(p_hbm.at[sl], table.at[sl])

                @pl.when(c == 1)
                def _():
                    pltpu.sync_copy(q_hbm.at[sl], table.at[sl])

        plsc.subcore_barrier()

        def chunk(k, which, out_hbm):
            pltpu.sync_copy(ei_hbm.at[which, base + k], idx_v)
            writes = [None, None]
            for j in range(_GSUB):
                b = j % 2
                if writes[b] is not None:
                    writes[b].wait()
                pltpu.async_copy(table.at[idx_v.at[j]], buf.at[b],
                                 sem_g).wait()
                writes[b] = pltpu.async_copy(
                    buf.at[b], out_hbm.at[pl.ds(k * _GCH + j * 128, 128)],
                    sem_w)
            writes[0].wait()
            writes[1].wait()

        @pl.loop(0, gmaxit)
        def _(jj):
            k = s + _NS * jj

            @pl.when(k < gnch)
            def _():
                @pl.when(c == 0)
                def _():
                    chunk(k, 0, ps_hbm)

                @pl.when(c == 1)
                def _():
                    chunk(k, 1, qr_hbm)

    return gather_kernel(p, q, ei3)


_SSUB = 5
_SCH = _SSUB * 128
_ZROWS = 80
_ZNCH = _N // _ZROWS
_ZMAXIT = (_ZNCH + _NS - 1) // _NS


def _sc_scatter_add(parts, ei3, zrows):
    nparts = len(parts)
    bases = [b for (_, b, _) in parts]
    snchs = [n for (_, _, n) in parts]
    mesh = plsc.VectorSubcoreMesh(core_axis_name="c", subcore_axis_name="s")

    @functools.partial(
        pl.kernel,
        mesh=mesh,
        out_type=jax.ShapeDtypeStruct((_NC, _N, _MSG), jnp.float32),
        scratch_types=[
            pltpu.VMEM((_SSUB, 128), jnp.int32),
            pltpu.VMEM((2, 128, _MSG), jnp.float32),
            pltpu.VMEM((_ZROWS, _MSG), jnp.float32),
            pltpu.VMEM_SHARED((_N, _MSG), jnp.float32),
            pltpu.SemaphoreType.DMA,
        ],
    )
    def scatter_kernel(*args):
        m_hbms = args[:nparts]
        ei_hbm, z_hbm, out_hbm, idx_v, mbuf, zbuf, acc, sem_m = args[nparts:]
        c = lax.axis_index("c")
        s = lax.axis_index("s")

        pltpu.sync_copy(z_hbm, zbuf)

        @pl.loop(0, _ZMAXIT)
        def _(jj):
            ch = s + _NS * jj

            @pl.when(ch < _ZNCH)
            def _():
                pltpu.sync_copy(zbuf, acc.at[pl.ds(ch * _ZROWS, _ZROWS)])

        plsc.subcore_barrier()

        for m_hbm, base, snch in zip(m_hbms, bases, snchs):
            hmaxit = (snch // 2 + _NS - 1) // _NS

            @pl.loop(0, hmaxit)
            def _(jj, m_hbm=m_hbm, base=base, snch=snch):
                t = 2 * (s + _NS * jj) + c

                @pl.when(t < snch)
                def _():
                    pltpu.sync_copy(ei_hbm.at[1, base + t], idx_v)
                    loads = [None, None]
                    loads[0] = pltpu.async_copy(
                        m_hbm.at[pl.ds(t * _SCH, 128)], mbuf.at[0], sem_m)
                    for j in range(_SSUB):
                        if j + 1 < _SSUB:
                            loads[(j + 1) % 2] = pltpu.async_copy(
                                m_hbm.at[pl.ds(t * _SCH + (j + 1) * 128, 128)],
                                mbuf.at[(j + 1) % 2], sem_m)
                        loads[j % 2].wait()
                        pltpu.sync_copy(mbuf.at[j % 2],
                                        acc.at[idx_v.at[j]], add=True)

        plsc.subcore_barrier()

        @pl.loop(0, _ZMAXIT)
        def _(jj):
            ch = s + _NS * jj

            @pl.when(ch < _ZNCH)
            def _():
                pltpu.sync_copy(acc.at[pl.ds(ch * _ZROWS, _ZROWS)],
                                out_hbm.at[c, pl.ds(ch * _ZROWS, _ZROWS)])

    return scatter_kernel(*[m for (m, _, _) in parts], ei3, zrows)


def _pack_pair(v16):
    vb = lax.bitcast_convert_type(v16, jnp.uint16)
    w = (vb[:, :128].astype(jnp.uint32)
         | (vb[:, 128:].astype(jnp.uint32) << 16))
    return lax.bitcast_convert_type(w, jnp.float32)


def _unpack_pair(wf):
    w = lax.bitcast_convert_type(wf, jnp.uint32)
    lo = lax.bitcast_convert_type((w & 0xFFFF).astype(jnp.uint16), jnp.bfloat16)
    hi = lax.bitcast_convert_type((w >> 16).astype(jnp.uint16), jnp.bfloat16)
    return lo, hi


def _pq_body(x_ref, w1a_ref, w1b_ref, b1_ref, p_ref, q_ref):
    xb = x_ref[...].astype(jnp.bfloat16)
    p = (jnp.dot(xb, w1a_ref[...], preferred_element_type=jnp.float32)
         + b1_ref[...])
    q = jnp.dot(xb, w1b_ref[...], preferred_element_type=jnp.float32)
    p_ref[...] = _pack_pair(p.astype(jnp.bfloat16))
    q_ref[...] = _pack_pair(q.astype(jnp.bfloat16))


def _edge_body(ps_ref, qr_ref, w2a_ref, w2b_ref, b2_ref, w3_ref, b3_ref, m_ref):
    ps_a, ps_b = _unpack_pair(ps_ref[...])
    qr_a, qr_b = _unpack_pair(qr_ref[...])
    h1a = jnp.maximum(ps_a + qr_a, 0)
    h1b = jnp.maximum(ps_b + qr_b, 0)
    h2 = (jnp.dot(h1a, w2a_ref[...], preferred_element_type=jnp.float32)
          + jnp.dot(h1b, w2b_ref[...], preferred_element_type=jnp.float32)
          + b2_ref[...])
    h2 = jnp.maximum(h2, 0.0).astype(jnp.bfloat16)
    m_ref[...] = (jnp.dot(h2, w3_ref[...], preferred_element_type=jnp.float32)
                  + b3_ref[...])


def _node_body(ag0_ref, ag1_ref, x_ref, wa1_ref, ba1_ref, wa2_ref, ba2_ref,
               wu1a_ref, wu1b_ref, bu1_ref, wu2_ref, bu2_ref,
               wu3_ref, bu3_ref, wh1_ref, bh1_ref, wh2_ref, bh2_ref,
               wm_ref, bm_ref, wl_ref, bl_ref, out_ref, act_ref):
    aggr = (jnp.sum(ag0_ref[...], axis=0)
            + jnp.sum(ag1_ref[...], axis=0)).astype(jnp.bfloat16)
    a = jnp.maximum(
        jnp.dot(aggr, wa1_ref[...], preferred_element_type=jnp.float32)
        + ba1_ref[...], 0.0).astype(jnp.bfloat16)
    a = jnp.maximum(
        jnp.dot(a, wa2_ref[...], preferred_element_type=jnp.float32)
        + ba2_ref[...], 0.0).astype(jnp.bfloat16)
    h = jnp.maximum(
        jnp.dot(x_ref[...].astype(jnp.bfloat16), wu1a_ref[...],
                preferred_element_type=jnp.float32)
        + jnp.dot(a, wu1b_ref[...], preferred_element_type=jnp.float32)
        + bu1_ref[...], 0.0).astype(jnp.bfloat16)
    h = jnp.maximum(
        jnp.dot(h, wu2_ref[...], preferred_element_type=jnp.float32)
        + bu2_ref[...], 0.0).astype(jnp.bfloat16)
    nodes = (jnp.dot(h, wu3_ref[...], preferred_element_type=jnp.float32)
             + bu3_ref[...])
    out_ref[...] = nodes

    @pl.when(pl.program_id(0) == 0)
    def _():
        z = jnp.maximum(
            jnp.dot(nodes[:_NA], wh1_ref[...],
                    preferred_element_type=jnp.float32) + bh1_ref[...], 0.0)
        z = jnp.maximum(
            jnp.dot(z, wh2_ref[...], preferred_element_type=jnp.float32)
            + bh2_ref[...], 0.0)
        mean = (jnp.dot(z, wm_ref[...], preferred_element_type=jnp.float32)
                + bm_ref[...])
        ls = jnp.clip(
            jnp.dot(z, wl_ref[...], preferred_element_type=jnp.float32)
            + bl_ref[...], -20.0, 2.0)
        act_ref[...] = jnp.concatenate([mean, jnp.exp(ls)], axis=-1)


def _full(shape):
    return pl.BlockSpec(shape, lambda *a: tuple(0 for _ in shape))


_BN = 2000
_BE = 2000


def _tc_pq(x, w1a, w1b, b1):
    grid = (_N // _BN,)
    return pl.pallas_call(
        _pq_body,
        grid=grid,
        in_specs=[
            pl.BlockSpec((_BN, _D), lambda i: (i, 0)),
            _full((_D, 256)),
            _full((_D, 256)),
            _full((1, 256)),
        ],
        out_specs=[
            pl.BlockSpec((_BN, _PKW), lambda i: (i, 0)),
            pl.BlockSpec((_BN, _PKW), lambda i: (i, 0)),
        ],
        out_shape=[
            jax.ShapeDtypeStruct((_N, _PKW), jnp.float32),
            jax.ShapeDtypeStruct((_N, _PKW), jnp.float32),
        ],
    )(x, w1a, w1b, b1)


def _tc_edge_mlp(ps, qr, w2a, w2b, b2, w3, b3):
    grid = (ps.shape[0] // _BE,)
    return pl.pallas_call(
        _edge_body,
        grid=grid,
        in_specs=[
            pl.BlockSpec((_BE, _PKW), lambda i: (i, 0)),
            pl.BlockSpec((_BE, _PKW), lambda i: (i, 0)),
            _full((128, 256)),
            _full((128, 256)),
            _full((1, 256)),
            _full((256, _MSG)),
            _full((1, _MSG)),
        ],
        out_specs=pl.BlockSpec((_BE, _MSG), lambda i: (i, 0)),
        out_shape=jax.ShapeDtypeStruct((ps.shape[0], _MSG), jnp.float32),
    )(ps, qr, w2a, w2b, b2, w3, b3)


def _tc_node_mlp(ag0, ag1, x, wa1, ba1, wa2, ba2, wu1a, wu1b, bu1, wu2, bu2,
                 wu3, bu3, wh1, bh1, wh2, bh2, wm, bm, wl, bl):
    grid = (_N // _BN,)
    return pl.pallas_call(
        _node_body,
        grid=grid,
        in_specs=[
            pl.BlockSpec((_NC, _BN, _MSG), lambda i: (0, i, 0)),
            pl.BlockSpec((_NC, _BN, _MSG), lambda i: (0, i, 0)),
            pl.BlockSpec((_BN, _D), lambda i: (i, 0)),
            _full((_MSG, 128)),
            _full((1, 128)),
            _full((128, 128)),
            _full((1, 128)),
            _full((_D, 256)),
            _full((128, 256)),
            _full((1, 256)),
            _full((256, 256)),
            _full((1, 256)),
            _full((256, _OUT)),
            _full((1, _OUT)),
            _full((_OUT, 256)),
            _full((1, 256)),
            _full((256, 256)),
            _full((1, 256)),
            _full((256, _ACT)),
            _full((1, _ACT)),
            _full((256, _ACT)),
            _full((1, _ACT)),
        ],
        out_specs=[
            pl.BlockSpec((_BN, _OUT), lambda i: (i, 0)),
            _full((_NA, 2 * _ACT)),
        ],
        out_shape=[
            jax.ShapeDtypeStruct((_N, _OUT), jnp.float32),
            jax.ShapeDtypeStruct((_NA, 2 * _ACT), jnp.float32),
        ],
    )(ag0, ag1, x, wa1, ba1, wa2, ba2, wu1a, wu1b, bu1, wu2, bu2, wu3, bu3,
      wh1, bh1, wh2, bh2, wm, bm, wl, bl)


def kernel(x, edge_index, Wm1, bm1, Wm2, bm2, Wm3, bm3, Wa1, ba1, Wa2, ba2,
           Wu1, bu1, Wu2, bu2, Wu3, bu3, Wh1, bh1, Wh2, bh2,
           Wmean, bmean, Wls, bls):
    ei3 = edge_index.reshape(2, _E // _GCH, _GSUB, 128)

    bf = jnp.bfloat16
    w1a, w1b = Wm1[:_D].astype(bf), Wm1[_D:].astype(bf)
    wu1a, wu1b = Wu1[:_D].astype(bf), Wu1[_D:].astype(bf)

    p, q = _tc_pq(x, w1a, w1b, bm1.reshape(1, -1))
    zrows = jnp.zeros((_ZROWS, _MSG), jnp.float32)
    w2a = Wm2[:128].astype(bf)
    w2b = Wm2[128:].astype(bf)
    w3 = Wm3.astype(bf)

    nch = _E // _GCH
    npart = 2
    pch = nch // npart
    ms = []
    for k in range(npart):
        ps, qr = _sc_gather(p, q, ei3, k * pch, pch)
        ms.append((_tc_edge_mlp(ps, qr, w2a, w2b, bm2.reshape(1, -1),
                                w3, bm3.reshape(1, -1)), k * pch, pch))
    ag0 = _sc_scatter_add(ms[:1], ei3, zrows)
    ag1 = _sc_scatter_add(ms[1:], ei3, zrows)
    nodes, act = _tc_node_mlp(
        ag0, ag1, x, Wa1.astype(bf), ba1.reshape(1, -1),
        Wa2.astype(bf), ba2.reshape(1, -1),
        wu1a, wu1b, bu1.reshape(1, -1),
        Wu2.astype(bf), bu2.reshape(1, -1),
        Wu3.astype(bf), bu3.reshape(1, -1),
        Wh1, bh1.reshape(1, -1), Wh2, bh2.reshape(1, -1),
        Wmean, bmean.reshape(1, -1), Wls, bls.reshape(1, -1))
    return act


# speedup vs baseline: 1.1944x; 1.0379x over previous
import functools

import jax
import jax.numpy as jnp
from jax import lax
from jax.experimental import pallas as pl
from jax.experimental.pallas import tpu as pltpu
from jax.experimental.pallas import tpu_sc as plsc

_N = 10000
_E = 160000
_D = 256
_MSG = 128
_OUT = 128
_ACT = 8
_NA = 1000

_NC = 2
_NS = 16
_NW = _NC * _NS


_GSUB = 5
_GCH = _GSUB * 128
_PKW = 128


def _sc_gather(p, q, ei3, base, gnch):
    ec = gnch * _GCH
    gmaxit = (gnch + _NS - 1) // _NS
    mesh = plsc.VectorSubcoreMesh(core_axis_name="c", subcore_axis_name="s")

    @functools.partial(
        pl.kernel,
        mesh=mesh,
        out_type=[
            jax.ShapeDtypeStruct((ec, _PKW), jnp.float32),
            jax.ShapeDtypeStruct((ec, _PKW), jnp.float32),
        ],
        scratch_types=[
            pltpu.VMEM_SHARED((_N, _PKW), jnp.float32),
            pltpu.VMEM((_GSUB, 128), jnp.int32),
            pltpu.VMEM((2, 128, _PKW), jnp.float32),
            pltpu.SemaphoreType.DMA,
            pltpu.SemaphoreType.DMA,
        ],
    )
    def gather_kernel(p_hbm, q_hbm, ei_hbm, ps_hbm, qr_hbm,
                      table, idx_v, buf, sem_g, sem_w):
        c = lax.axis_index("c")
        s = lax.axis_index("s")

        @pl.loop(0, _ZMAXIT)
        def _(jj):
            ch = s + _NS * jj

            @pl.when(ch < _ZNCH)
            def _():
                sl = pl.ds(ch * _ZROWS, _ZROWS)

                @pl.when(c == 0)
                def _():
                    pltpu.sync_copy(p_hbm.at[sl], table.at[sl])

                @pl.when(c == 1)
                def _():
                    pltpu.sync_copy(q_hbm.at[sl], table.at[sl])

        plsc.subcore_barrier()

        def chunk(k, which, out_hbm):
            pltpu.sync_copy(ei_hbm.at[which, base + k], idx_v)
            writes = [None, None]
            for j in range(_GSUB):
                b = j % 2
                if writes[b] is not None:
                    writes[b].wait()
                pltpu.async_copy(table.at[idx_v.at[j]], buf.at[b],
                                 sem_g).wait()
                writes[b] = pltpu.async_copy(
                    buf.at[b], out_hbm.at[pl.ds(k * _GCH + j * 128, 128)],
                    sem_w)
            writes[0].wait()
            writes[1].wait()

        @pl.loop(0, gmaxit)
        def _(jj):
            k = s + _NS * jj

            @pl.when(k < gnch)
            def _():
                @pl.when(c == 0)
                def _():
                    chunk(k, 0, ps_hbm)

                @pl.when(c == 1)
                def _():
                    chunk(k, 1, qr_hbm)

    return gather_kernel(p, q, ei3)


_SSUB = 5
_SCH = _SSUB * 128
_ZROWS = 80
_ZNCH = _N // _ZROWS
_ZMAXIT = (_ZNCH + _NS - 1) // _NS


def _sc_scatter_add(parts, ei3, zrows):
    nparts = len(parts)
    bases = [b for (_, b, _) in parts]
    snchs = [n for (_, _, n) in parts]
    mesh = plsc.VectorSubcoreMesh(core_axis_name="c", subcore_axis_name="s")

    @functools.partial(
        pl.kernel,
        mesh=mesh,
        out_type=jax.ShapeDtypeStruct((_NC, _N, _MSG), jnp.float32),
        scratch_types=[
            pltpu.VMEM((_SSUB, 128), jnp.int32),
            pltpu.VMEM((2, 128, _MSG), jnp.float32),
            pltpu.VMEM((_ZROWS, _MSG), jnp.float32),
            pltpu.VMEM_SHARED((_N, _MSG), jnp.float32),
            pltpu.SemaphoreType.DMA,
        ],
    )
    def scatter_kernel(*args):
        m_hbms = args[:nparts]
        ei_hbm, z_hbm, out_hbm, idx_v, mbuf, zbuf, acc, sem_m = args[nparts:]
        c = lax.axis_index("c")
        s = lax.axis_index("s")

        pltpu.sync_copy(z_hbm, zbuf)

        @pl.loop(0, _ZMAXIT)
        def _(jj):
            ch = s + _NS * jj

            @pl.when(ch < _ZNCH)
            def _():
                pltpu.sync_copy(zbuf, acc.at[pl.ds(ch * _ZROWS, _ZROWS)])

        plsc.subcore_barrier()

        for m_hbm, base, snch in zip(m_hbms, bases, snchs):
            hmaxit = (snch // 2 + _NS - 1) // _NS

            @pl.loop(0, hmaxit)
            def _(jj, m_hbm=m_hbm, base=base, snch=snch):
                t = 2 * (s + _NS * jj) + c

                @pl.when(t < snch)
                def _():
                    pltpu.sync_copy(ei_hbm.at[1, base + t], idx_v)
                    loads = [None, None]
                    loads[0] = pltpu.async_copy(
                        m_hbm.at[pl.ds(t * _SCH, 128)], mbuf.at[0], sem_m)
                    for j in range(_SSUB):
                        if j + 1 < _SSUB:
                            loads[(j + 1) % 2] = pltpu.async_copy(
                                m_hbm.at[pl.ds(t * _SCH + (j + 1) * 128, 128)],
                                mbuf.at[(j + 1) % 2], sem_m)
                        loads[j % 2].wait()
                        pltpu.sync_copy(mbuf.at[j % 2],
                                        acc.at[idx_v.at[j]], add=True)

        plsc.subcore_barrier()

        @pl.loop(0, _ZMAXIT)
        def _(jj):
            ch = s + _NS * jj

            @pl.when(ch < _ZNCH)
            def _():
                pltpu.sync_copy(acc.at[pl.ds(ch * _ZROWS, _ZROWS)],
                                out_hbm.at[c, pl.ds(ch * _ZROWS, _ZROWS)])

    return scatter_kernel(*[m for (m, _, _) in parts], ei3, zrows)


def _pack_pair(v16):
    vb = lax.bitcast_convert_type(v16, jnp.uint16)
    w = (vb[:, :128].astype(jnp.uint32)
         | (vb[:, 128:].astype(jnp.uint32) << 16))
    return lax.bitcast_convert_type(w, jnp.float32)


def _unpack_pair(wf):
    w = lax.bitcast_convert_type(wf, jnp.uint32)
    lo = lax.bitcast_convert_type((w & 0xFFFF).astype(jnp.uint16), jnp.bfloat16)
    hi = lax.bitcast_convert_type((w >> 16).astype(jnp.uint16), jnp.bfloat16)
    return lo, hi


def _pq_body(x_ref, w1a_ref, w1b_ref, b1_ref, p_ref, q_ref):
    xb = x_ref[...].astype(jnp.bfloat16)
    p = (jnp.dot(xb, w1a_ref[...], preferred_element_type=jnp.float32)
         + b1_ref[...])
    q = jnp.dot(xb, w1b_ref[...], preferred_element_type=jnp.float32)
    p_ref[...] = _pack_pair(p.astype(jnp.bfloat16))
    q_ref[...] = _pack_pair(q.astype(jnp.bfloat16))


def _edge_body(ps_ref, qr_ref, w2a_ref, w2b_ref, b2_ref, w3_ref, b3_ref, m_ref):
    ps_a, ps_b = _unpack_pair(ps_ref[...])
    qr_a, qr_b = _unpack_pair(qr_ref[...])
    h1a = jnp.maximum(ps_a + qr_a, 0)
    h1b = jnp.maximum(ps_b + qr_b, 0)
    h2 = (jnp.dot(h1a, w2a_ref[...], preferred_element_type=jnp.float32)
          + jnp.dot(h1b, w2b_ref[...], preferred_element_type=jnp.float32)
          + b2_ref[...])
    h2 = jnp.maximum(h2, 0.0).astype(jnp.bfloat16)
    m_ref[...] = (jnp.dot(h2, w3_ref[...], preferred_element_type=jnp.float32)
                  + b3_ref[...])


def _node_body(ag0_ref, ag1_ref, x_ref, wa1_ref, ba1_ref, wa2_ref, ba2_ref,
               wu1a_ref, wu1b_ref, bu1_ref, wu2_ref, bu2_ref,
               wu3_ref, bu3_ref, wh1_ref, bh1_ref, wh2_ref, bh2_ref,
               wm_ref, bm_ref, wl_ref, bl_ref, out_ref, act_ref):
    aggr = (jnp.sum(ag0_ref[...], axis=0)
            + jnp.sum(ag1_ref[...], axis=0)).astype(jnp.bfloat16)
    a = jnp.maximum(
        jnp.dot(aggr, wa1_ref[...], preferred_element_type=jnp.float32)
        + ba1_ref[...], 0.0).astype(jnp.bfloat16)
    a = jnp.maximum(
        jnp.dot(a, wa2_ref[...], preferred_element_type=jnp.float32)
        + ba2_ref[...], 0.0).astype(jnp.bfloat16)
    h = jnp.maximum(
        jnp.dot(x_ref[...].astype(jnp.bfloat16), wu1a_ref[...],
                preferred_element_type=jnp.float32)
        + jnp.dot(a, wu1b_ref[...], preferred_element_type=jnp.float32)
        + bu1_ref[...], 0.0).astype(jnp.bfloat16)
    h = jnp.maximum(
        jnp.dot(h, wu2_ref[...], preferred_element_type=jnp.float32)
        + bu2_ref[...], 0.0).astype(jnp.bfloat16)
    nodes = (jnp.dot(h, wu3_ref[...], preferred_element_type=jnp.float32)
             + bu3_ref[...])
    out_ref[...] = nodes

    @pl.when(pl.program_id(0) == 0)
    def _():
        z = jnp.maximum(
            jnp.dot(nodes[:_NA], wh1_ref[...],
                    preferred_element_type=jnp.float32) + bh1_ref[...], 0.0)
        z = jnp.maximum(
            jnp.dot(z, wh2_ref[...], preferred_element_type=jnp.float32)
            + bh2_ref[...], 0.0)
        mean = (jnp.dot(z, wm_ref[...], preferred_element_type=jnp.float32)
                + bm_ref[...])
        ls = jnp.clip(
            jnp.dot(z, wl_ref[...], preferred_element_type=jnp.float32)
            + bl_ref[...], -20.0, 2.0)
        act_ref[...] = jnp.concatenate([mean, jnp.exp(ls)], axis=-1)


def _full(shape):
    return pl.BlockSpec(shape, lambda *a: tuple(0 for _ in shape))


_BN = 2000
_BE = 4000


def _tc_pq(x, w1a, w1b, b1):
    grid = (_N // _BN,)
    return pl.pallas_call(
        _pq_body,
        grid=grid,
        in_specs=[
            pl.BlockSpec((_BN, _D), lambda i: (i, 0)),
            _full((_D, 256)),
            _full((_D, 256)),
            _full((1, 256)),
        ],
        out_specs=[
            pl.BlockSpec((_BN, _PKW), lambda i: (i, 0)),
            pl.BlockSpec((_BN, _PKW), lambda i: (i, 0)),
        ],
        out_shape=[
            jax.ShapeDtypeStruct((_N, _PKW), jnp.float32),
            jax.ShapeDtypeStruct((_N, _PKW), jnp.float32),
        ],
    )(x, w1a, w1b, b1)


def _tc_edge_mlp(ps, qr, w2a, w2b, b2, w3, b3):
    grid = (ps.shape[0] // _BE,)
    return pl.pallas_call(
        _edge_body,
        grid=grid,
        in_specs=[
            pl.BlockSpec((_BE, _PKW), lambda i: (i, 0)),
            pl.BlockSpec((_BE, _PKW), lambda i: (i, 0)),
            _full((128, 256)),
            _full((128, 256)),
            _full((1, 256)),
            _full((256, _MSG)),
            _full((1, _MSG)),
        ],
        out_specs=pl.BlockSpec((_BE, _MSG), lambda i: (i, 0)),
        out_shape=jax.ShapeDtypeStruct((ps.shape[0], _MSG), jnp.float32),
    )(ps, qr, w2a, w2b, b2, w3, b3)


def _tc_node_mlp(ag0, ag1, x, wa1, ba1, wa2, ba2, wu1a, wu1b, bu1, wu2, bu2,
                 wu3, bu3, wh1, bh1, wh2, bh2, wm, bm, wl, bl):
    grid = (_N // _BN,)
    return pl.pallas_call(
        _node_body,
        grid=grid,
        in_specs=[
            pl.BlockSpec((_NC, _BN, _MSG), lambda i: (0, i, 0)),
            pl.BlockSpec((_NC, _BN, _MSG), lambda i: (0, i, 0)),
            pl.BlockSpec((_BN, _D), lambda i: (i, 0)),
            _full((_MSG, 128)),
            _full((1, 128)),
            _full((128, 128)),
            _full((1, 128)),
            _full((_D, 256)),
            _full((128, 256)),
            _full((1, 256)),
            _full((256, 256)),
            _full((1, 256)),
            _full((256, _OUT)),
            _full((1, _OUT)),
            _full((_OUT, 256)),
            _full((1, 256)),
            _full((256, 256)),
            _full((1, 256)),
            _full((256, _ACT)),
            _full((1, _ACT)),
            _full((256, _ACT)),
            _full((1, _ACT)),
        ],
        out_specs=[
            pl.BlockSpec((_BN, _OUT), lambda i: (i, 0)),
            _full((_NA, 2 * _ACT)),
        ],
        out_shape=[
            jax.ShapeDtypeStruct((_N, _OUT), jnp.float32),
            jax.ShapeDtypeStruct((_NA, 2 * _ACT), jnp.float32),
        ],
    )(ag0, ag1, x, wa1, ba1, wa2, ba2, wu1a, wu1b, bu1, wu2, bu2, wu3, bu3,
      wh1, bh1, wh2, bh2, wm, bm, wl, bl)


def kernel(x, edge_index, Wm1, bm1, Wm2, bm2, Wm3, bm3, Wa1, ba1, Wa2, ba2,
           Wu1, bu1, Wu2, bu2, Wu3, bu3, Wh1, bh1, Wh2, bh2,
           Wmean, bmean, Wls, bls):
    ei3 = edge_index.reshape(2, _E // _GCH, _GSUB, 128)

    bf = jnp.bfloat16
    w1a, w1b = Wm1[:_D].astype(bf), Wm1[_D:].astype(bf)
    wu1a, wu1b = Wu1[:_D].astype(bf), Wu1[_D:].astype(bf)

    p, q = _tc_pq(x, w1a, w1b, bm1.reshape(1, -1))
    zrows = jnp.zeros((_ZROWS, _MSG), jnp.float32)
    w2a = Wm2[:128].astype(bf)
    w2b = Wm2[128:].astype(bf)
    w3 = Wm3.astype(bf)

    nch = _E // _GCH
    npart = 2
    pch = nch // npart
    ms = []
    for k in range(npart):
        ps, qr = _sc_gather(p, q, ei3, k * pch, pch)
        ms.append((_tc_edge_mlp(ps, qr, w2a, w2b, bm2.reshape(1, -1),
                                w3, bm3.reshape(1, -1)), k * pch, pch))
    ag0 = _sc_scatter_add(ms[:1], ei3, zrows)
    ag1 = _sc_scatter_add(ms[1:], ei3, zrows)
    nodes, act = _tc_node_mlp(
        ag0, ag1, x, Wa1.astype(bf), ba1.reshape(1, -1),
        Wa2.astype(bf), ba2.reshape(1, -1),
        wu1a, wu1b, bu1.reshape(1, -1),
        Wu2.astype(bf), bu2.reshape(1, -1),
        Wu3.astype(bf), bu3.reshape(1, -1),
        Wh1, bh1.reshape(1, -1), Wh2, bh2.reshape(1, -1),
        Wmean, bmean.reshape(1, -1), Wls, bls.reshape(1, -1))
    return act


# speedup vs baseline: 1.2024x; 1.0067x over previous
import functools

import jax
import jax.numpy as jnp
from jax import lax
from jax.experimental import pallas as pl
from jax.experimental.pallas import tpu as pltpu
from jax.experimental.pallas import tpu_sc as plsc

_N = 10000
_E = 160000
_D = 256
_MSG = 128
_OUT = 128
_ACT = 8
_NA = 1000

_NC = 2
_NS = 16
_NW = _NC * _NS


_GSUB = 5
_GCH = _GSUB * 128
_PKW = 128


def _sc_gather(p, q, ei3, base, gnch):
    ec = gnch * _GCH
    gmaxit = (gnch + _NS - 1) // _NS
    mesh = plsc.VectorSubcoreMesh(core_axis_name="c", subcore_axis_name="s")

    @functools.partial(
        pl.kernel,
        mesh=mesh,
        out_type=[
            jax.ShapeDtypeStruct((ec, _PKW), jnp.float32),
            jax.ShapeDtypeStruct((ec, _PKW), jnp.float32),
        ],
        scratch_types=[
            pltpu.VMEM_SHARED((_N, _PKW), jnp.float32),
            pltpu.VMEM((_GSUB, 128), jnp.int32),
            pltpu.VMEM((2, 128, _PKW), jnp.float32),
            pltpu.SemaphoreType.DMA,
            pltpu.SemaphoreType.DMA,
        ],
    )
    def gather_kernel(p_hbm, q_hbm, ei_hbm, ps_hbm, qr_hbm,
                      table, idx_v, buf, sem_g, sem_w):
        c = lax.axis_index("c")
        s = lax.axis_index("s")

        @pl.loop(0, _ZMAXIT)
        def _(jj):
            ch = s + _NS * jj

            @pl.when(ch < _ZNCH)
            def _():
                sl = pl.ds(ch * _ZROWS, _ZROWS)

                @pl.when(c == 0)
                def _():
                    pltpu.sync_copy(p_hbm.at[sl], table.at[sl])

                @pl.when(c == 1)
                def _():
                    pltpu.sync_copy(q_hbm.at[sl], table.at[sl])

        plsc.subcore_barrier()

        def chunk(k, which, out_hbm):
            pltpu.sync_copy(ei_hbm.at[which, base + k], idx_v)
            writes = [None, None]
            for j in range(_GSUB):
                b = j % 2
                if writes[b] is not None:
                    writes[b].wait()
                pltpu.async_copy(table.at[idx_v.at[j]], buf.at[b],
                                 sem_g).wait()
                writes[b] = pltpu.async_copy(
                    buf.at[b], out_hbm.at[pl.ds(k * _GCH + j * 128, 128)],
                    sem_w)
            writes[0].wait()
            writes[1].wait()

        @pl.loop(0, gmaxit)
        def _(jj):
            k = s + _NS * jj

            @pl.when(k < gnch)
            def _():
                @pl.when(c == 0)
                def _():
                    chunk(k, 0, ps_hbm)

                @pl.when(c == 1)
                def _():
                    chunk(k, 1, qr_hbm)

    return gather_kernel(p, q, ei3)


_SSUB = 5
_SCH = _SSUB * 128
_ZROWS = 80
_ZNCH = _N // _ZROWS
_ZMAXIT = (_ZNCH + _NS - 1) // _NS


def _sc_scatter_add(parts, ei3, zrows):
    nparts = len(parts)
    bases = [b for (_, b, _) in parts]
    snchs = [n for (_, _, n) in parts]
    mesh = plsc.VectorSubcoreMesh(core_axis_name="c", subcore_axis_name="s")

    @functools.partial(
        pl.kernel,
        mesh=mesh,
        out_type=jax.ShapeDtypeStruct((_NC, _N, _MSG), jnp.float32),
        scratch_types=[
            pltpu.VMEM((_SSUB, 128), jnp.int32),
            pltpu.VMEM((2, 128, _MSG), jnp.float32),
            pltpu.VMEM((_ZROWS, _MSG), jnp.float32),
            pltpu.VMEM_SHARED((_N, _MSG), jnp.float32),
            pltpu.SemaphoreType.DMA,
        ],
    )
    def scatter_kernel(*args):
        m_hbms = args[:nparts]
        ei_hbm, z_hbm, out_hbm, idx_v, mbuf, zbuf, acc, sem_m = args[nparts:]
        c = lax.axis_index("c")
        s = lax.axis_index("s")

        pltpu.sync_copy(z_hbm, zbuf)

        @pl.loop(0, _ZMAXIT)
        def _(jj):
            ch = s + _NS * jj

            @pl.when(ch < _ZNCH)
            def _():
                pltpu.sync_copy(zbuf, acc.at[pl.ds(ch * _ZROWS, _ZROWS)])

        plsc.subcore_barrier()

        for m_hbm, base, snch in zip(m_hbms, bases, snchs):
            hmaxit = (snch // 2 + _NS - 1) // _NS

            @pl.loop(0, hmaxit)
            def _(jj, m_hbm=m_hbm, base=base, snch=snch):
                t = 2 * (s + _NS * jj) + c

                @pl.when(t < snch)
                def _():
                    pltpu.sync_copy(ei_hbm.at[1, base + t], idx_v)
                    loads = [None, None]
                    loads[0] = pltpu.async_copy(
                        m_hbm.at[pl.ds(t * _SCH, 128)], mbuf.at[0], sem_m)
                    for j in range(_SSUB):
                        if j + 1 < _SSUB:
                            loads[(j + 1) % 2] = pltpu.async_copy(
                                m_hbm.at[pl.ds(t * _SCH + (j + 1) * 128, 128)],
                                mbuf.at[(j + 1) % 2], sem_m)
                        loads[j % 2].wait()
                        pltpu.sync_copy(mbuf.at[j % 2],
                                        acc.at[idx_v.at[j]], add=True)

        plsc.subcore_barrier()

        @pl.loop(0, _ZMAXIT)
        def _(jj):
            ch = s + _NS * jj

            @pl.when(ch < _ZNCH)
            def _():
                pltpu.sync_copy(acc.at[pl.ds(ch * _ZROWS, _ZROWS)],
                                out_hbm.at[c, pl.ds(ch * _ZROWS, _ZROWS)])

    return scatter_kernel(*[m for (m, _, _) in parts], ei3, zrows)


def _pack_pair(v16):
    vb = lax.bitcast_convert_type(v16, jnp.uint16)
    w = (vb[:, :128].astype(jnp.uint32)
         | (vb[:, 128:].astype(jnp.uint32) << 16))
    return lax.bitcast_convert_type(w, jnp.float32)


def _unpack_pair(wf):
    w = lax.bitcast_convert_type(wf, jnp.uint32)
    lo = lax.bitcast_convert_type((w & 0xFFFF).astype(jnp.uint16), jnp.bfloat16)
    hi = lax.bitcast_convert_type((w >> 16).astype(jnp.uint16), jnp.bfloat16)
    return lo, hi


def _pq_body(x_ref, w1a_ref, w1b_ref, b1_ref, p_ref, q_ref):
    xb = x_ref[...].astype(jnp.bfloat16)
    p = (jnp.dot(xb, w1a_ref[...], preferred_element_type=jnp.float32)
         + b1_ref[...])
    q = jnp.dot(xb, w1b_ref[...], preferred_element_type=jnp.float32)
    p_ref[...] = _pack_pair(p.astype(jnp.bfloat16))
    q_ref[...] = _pack_pair(q.astype(jnp.bfloat16))


def _edge_body(ps_ref, qr_ref, w2a_ref, w2b_ref, b2_ref, w3_ref, b3_ref, m_ref):
    ps_a, ps_b = _unpack_pair(ps_ref[...])
    qr_a, qr_b = _unpack_pair(qr_ref[...])
    h1a = jnp.maximum(ps_a + qr_a, 0)
    h1b = jnp.maximum(ps_b + qr_b, 0)
    h2 = (jnp.dot(h1a, w2a_ref[...], preferred_element_type=jnp.float32)
          + jnp.dot(h1b, w2b_ref[...], preferred_element_type=jnp.float32)
          + b2_ref[...])
    h2 = jnp.maximum(h2, 0.0).astype(jnp.bfloat16)
    m_ref[...] = (jnp.dot(h2, w3_ref[...], preferred_element_type=jnp.float32)
                  + b3_ref[...])


def _node_body(ag0_ref, ag1_ref, x_ref, wa1_ref, ba1_ref, wa2_ref, ba2_ref,
               wu1a_ref, wu1b_ref, bu1_ref, wu2_ref, bu2_ref,
               wu3_ref, bu3_ref, wh1_ref, bh1_ref, wh2_ref, bh2_ref,
               wm_ref, bm_ref, wl_ref, bl_ref, out_ref, act_ref):
    aggr = (jnp.sum(ag0_ref[...], axis=0)
            + jnp.sum(ag1_ref[...], axis=0)).astype(jnp.bfloat16)
    a = jnp.maximum(
        jnp.dot(aggr, wa1_ref[...], preferred_element_type=jnp.float32)
        + ba1_ref[...], 0.0).astype(jnp.bfloat16)
    a = jnp.maximum(
        jnp.dot(a, wa2_ref[...], preferred_element_type=jnp.float32)
        + ba2_ref[...], 0.0).astype(jnp.bfloat16)
    h = jnp.maximum(
        jnp.dot(x_ref[...].astype(jnp.bfloat16), wu1a_ref[...],
                preferred_element_type=jnp.float32)
        + jnp.dot(a, wu1b_ref[...], preferred_element_type=jnp.float32)
        + bu1_ref[...], 0.0).astype(jnp.bfloat16)
    h = jnp.maximum(
        jnp.dot(h, wu2_ref[...], preferred_element_type=jnp.float32)
        + bu2_ref[...], 0.0).astype(jnp.bfloat16)
    nodes = (jnp.dot(h, wu3_ref[...], preferred_element_type=jnp.float32)
             + bu3_ref[...])
    out_ref[...] = nodes

    @pl.when(pl.program_id(0) == 0)
    def _():
        z = jnp.maximum(
            jnp.dot(nodes[:_NA], wh1_ref[...],
                    preferred_element_type=jnp.float32) + bh1_ref[...], 0.0)
        z = jnp.maximum(
            jnp.dot(z, wh2_ref[...], preferred_element_type=jnp.float32)
            + bh2_ref[...], 0.0)
        mean = (jnp.dot(z, wm_ref[...], preferred_element_type=jnp.float32)
                + bm_ref[...])
        ls = jnp.clip(
            jnp.dot(z, wl_ref[...], preferred_element_type=jnp.float32)
            + bl_ref[...], -20.0, 2.0)
        act_ref[...] = jnp.concatenate([mean, jnp.exp(ls)], axis=-1)


def _full(shape):
    return pl.BlockSpec(shape, lambda *a: tuple(0 for _ in shape))


_BN = 2000
_BE = 8000


def _tc_pq(x, w1a, w1b, b1):
    grid = (_N // _BN,)
    return pl.pallas_call(
        _pq_body,
        grid=grid,
        in_specs=[
            pl.BlockSpec((_BN, _D), lambda i: (i, 0)),
            _full((_D, 256)),
            _full((_D, 256)),
            _full((1, 256)),
        ],
        out_specs=[
            pl.BlockSpec((_BN, _PKW), lambda i: (i, 0)),
            pl.BlockSpec((_BN, _PKW), lambda i: (i, 0)),
        ],
        out_shape=[
            jax.ShapeDtypeStruct((_N, _PKW), jnp.float32),
            jax.ShapeDtypeStruct((_N, _PKW), jnp.float32),
        ],
    )(x, w1a, w1b, b1)


def _tc_edge_mlp(ps, qr, w2a, w2b, b2, w3, b3):
    grid = (ps.shape[0] // _BE,)
    return pl.pallas_call(
        _edge_body,
        grid=grid,
        in_specs=[
            pl.BlockSpec((_BE, _PKW), lambda i: (i, 0)),
            pl.BlockSpec((_BE, _PKW), lambda i: (i, 0)),
            _full((128, 256)),
            _full((128, 256)),
            _full((1, 256)),
            _full((256, _MSG)),
            _full((1, _MSG)),
        ],
        out_specs=pl.BlockSpec((_BE, _MSG), lambda i: (i, 0)),
        out_shape=jax.ShapeDtypeStruct((ps.shape[0], _MSG), jnp.float32),
    )(ps, qr, w2a, w2b, b2, w3, b3)


def _tc_node_mlp(ag0, ag1, x, wa1, ba1, wa2, ba2, wu1a, wu1b, bu1, wu2, bu2,
                 wu3, bu3, wh1, bh1, wh2, bh2, wm, bm, wl, bl):
    grid = (_N // _BN,)
    return pl.pallas_call(
        _node_body,
        grid=grid,
        in_specs=[
            pl.BlockSpec((_NC, _BN, _MSG), lambda i: (0, i, 0)),
            pl.BlockSpec((_NC, _BN, _MSG), lambda i: (0, i, 0)),
            pl.BlockSpec((_BN, _D), lambda i: (i, 0)),
            _full((_MSG, 128)),
            _full((1, 128)),
            _full((128, 128)),
            _full((1, 128)),
            _full((_D, 256)),
            _full((128, 256)),
            _full((1, 256)),
            _full((256, 256)),
            _full((1, 256)),
            _full((256, _OUT)),
            _full((1, _OUT)),
            _full((_OUT, 256)),
            _full((1, 256)),
            _full((256, 256)),
            _full((1, 256)),
            _full((256, _ACT)),
            _full((1, _ACT)),
            _full((256, _ACT)),
            _full((1, _ACT)),
        ],
        out_specs=[
            pl.BlockSpec((_BN, _OUT), lambda i: (i, 0)),
            _full((_NA, 2 * _ACT)),
        ],
        out_shape=[
            jax.ShapeDtypeStruct((_N, _OUT), jnp.float32),
            jax.ShapeDtypeStruct((_NA, 2 * _ACT), jnp.float32),
        ],
    )(ag0, ag1, x, wa1, ba1, wa2, ba2, wu1a, wu1b, bu1, wu2, bu2, wu3, bu3,
      wh1, bh1, wh2, bh2, wm, bm, wl, bl)


def kernel(x, edge_index, Wm1, bm1, Wm2, bm2, Wm3, bm3, Wa1, ba1, Wa2, ba2,
           Wu1, bu1, Wu2, bu2, Wu3, bu3, Wh1, bh1, Wh2, bh2,
           Wmean, bmean, Wls, bls):
    ei3 = edge_index.reshape(2, _E // _GCH, _GSUB, 128)

    bf = jnp.bfloat16
    w1a, w1b = Wm1[:_D].astype(bf), Wm1[_D:].astype(bf)
    wu1a, wu1b = Wu1[:_D].astype(bf), Wu1[_D:].astype(bf)

    p, q = _tc_pq(x, w1a, w1b, bm1.reshape(1, -1))
    zrows = jnp.zeros((_ZROWS, _MSG), jnp.float32)
    w2a = Wm2[:128].astype(bf)
    w2b = Wm2[128:].astype(bf)
    w3 = Wm3.astype(bf)

    nch = _E // _GCH
    npart = 2
    pch = nch // npart
    ms = []
    for k in range(npart):
        ps, qr = _sc_gather(p, q, ei3, k * pch, pch)
        ms.append((_tc_edge_mlp(ps, qr, w2a, w2b, bm2.reshape(1, -1),
                                w3, bm3.reshape(1, -1)), k * pch, pch))
    ag0 = _sc_scatter_add(ms[:1], ei3, zrows)
    ag1 = _sc_scatter_add(ms[1:], ei3, zrows)
    nodes, act = _tc_node_mlp(
        ag0, ag1, x, Wa1.astype(bf), ba1.reshape(1, -1),
        Wa2.astype(bf), ba2.reshape(1, -1),
        wu1a, wu1b, bu1.reshape(1, -1),
        Wu2.astype(bf), bu2.reshape(1, -1),
        Wu3.astype(bf), bu3.reshape(1, -1),
        Wh1, bh1.reshape(1, -1), Wh2, bh2.reshape(1, -1),
        Wmean, bmean.reshape(1, -1), Wls, bls.reshape(1, -1))
    return act
